# Initial kernel scaffold; baseline (speedup 1.0000x reference)
import jax
import jax.numpy as jnp
from jax.experimental import pallas as pl


def kernel(user_classes, table, W1, b1, g1, be1, W2, b2, g2, be2):
    raise NotImplementedError("write your pallas kernel here")


# speedup vs baseline: 2.5544x; 2.5544x over previous
import functools

import jax
import jax.numpy as jnp
from jax import lax
from jax.experimental import pallas as pl
from jax.experimental.pallas import tpu as pltpu
from jax.experimental.pallas import tpu_sc as plsc

NUM_USERS = 100000
EMBED_DIM = 1152
HIDDEN_DIM = 512
BATCH = 4096

_NC = 2
_NS = 16
_NW = _NC * _NS
_B_PER_W = BATCH // _NW
_CHUNK = 64


def _sc_gather(table, idx):
    @functools.partial(
        pl.kernel,
        mesh=plsc.VectorSubcoreMesh(core_axis_name="c", subcore_axis_name="s"),
        out_type=jax.ShapeDtypeStruct((BATCH, EMBED_DIM), jnp.float32),
        scratch_types=[
            pltpu.VMEM((_B_PER_W,), jnp.int32),
            pltpu.VMEM((_CHUNK, EMBED_DIM), jnp.float32),
            pltpu.SemaphoreType.DMA,
        ],
    )
    def k(table_hbm, idx_hbm, out_hbm, idx_v, rows_v, sem):
        wid = lax.axis_index("s") * _NC + lax.axis_index("c")
        base = wid * _B_PER_W
        pltpu.sync_copy(idx_hbm.at[pl.ds(base, _B_PER_W)], idx_v)
        for c in range(_B_PER_W // _CHUNK):
            off = c * _CHUNK
            pltpu.async_copy(
                table_hbm.at[idx_v.at[pl.ds(off, _CHUNK)]], rows_v, sem
            ).wait()
            pltpu.sync_copy(rows_v, out_hbm.at[pl.ds(base + off, _CHUNK)])

    return k(table, idx)


def _mlp_body(x_ref, w1_ref, b1_ref, g1_ref, be1_ref, w2_ref, b2_ref,
              g2_ref, be2_ref, o_ref):
    x = x_ref[...]
    h = jnp.dot(x, w1_ref[...], preferred_element_type=jnp.float32)
    h = h + b1_ref[...]
    mu = jnp.mean(h, axis=-1, keepdims=True)
    var = jnp.mean((h - mu) * (h - mu), axis=-1, keepdims=True)
    h = (h - mu) * lax.rsqrt(var + 1e-5) * g1_ref[...] + be1_ref[...]
    h = jnp.maximum(h, 0.0)
    y = jnp.dot(h, w2_ref[...], preferred_element_type=jnp.float32)
    y = y + b2_ref[...]
    mu2 = jnp.mean(y, axis=-1, keepdims=True)
    var2 = jnp.mean((y - mu2) * (y - mu2), axis=-1, keepdims=True)
    o_ref[...] = (y - mu2) * lax.rsqrt(var2 + 1e-5) * g2_ref[...] + be2_ref[...]


def _tc_mlp(x, W1, b1, g1, be1, W2, b2, g2, be2):
    BT = 256
    grid = (BATCH // BT,)
    full = lambda shape: pl.BlockSpec(shape, lambda i: (0, 0))
    return pl.pallas_call(
        _mlp_body,
        grid=grid,
        in_specs=[
            pl.BlockSpec((BT, EMBED_DIM), lambda i: (i, 0)),
            full((EMBED_DIM, HIDDEN_DIM)),
            full((1, HIDDEN_DIM)),
            full((1, HIDDEN_DIM)),
            full((1, HIDDEN_DIM)),
            full((HIDDEN_DIM, EMBED_DIM)),
            full((1, EMBED_DIM)),
            full((1, EMBED_DIM)),
            full((1, EMBED_DIM)),
        ],
        out_specs=pl.BlockSpec((BT, EMBED_DIM), lambda i: (i, 0)),
        out_shape=jax.ShapeDtypeStruct((BATCH, EMBED_DIM), jnp.float32),
        compiler_params=pltpu.CompilerParams(
            dimension_semantics=("parallel",),
        ),
    )(x, W1, b1.reshape(1, -1), g1.reshape(1, -1), be1.reshape(1, -1),
      W2, b2.reshape(1, -1), g2.reshape(1, -1), be2.reshape(1, -1))


def kernel(user_classes, table, W1, b1, g1, be1, W2, b2, g2, be2):
    gathered = _sc_gather(table, user_classes.astype(jnp.int32))
    return _tc_mlp(gathered, W1, b1, g1, be1, W2, b2, g2, be2)


# speedup vs baseline: 2.5678x; 1.0053x over previous
import functools

import jax
import jax.numpy as jnp
from jax import lax
from jax.experimental import pallas as pl
from jax.experimental.pallas import tpu as pltpu
from jax.experimental.pallas import tpu_sc as plsc

NUM_USERS = 100000
EMBED_DIM = 1152
HIDDEN_DIM = 512
BATCH = 4096

_NC = 2
_NS = 16
_NW = _NC * _NS
_B_PER_W = BATCH // _NW
_CHUNK = 64


def _sc_gather(table, idx):
    @functools.partial(
        pl.kernel,
        mesh=plsc.VectorSubcoreMesh(core_axis_name="c", subcore_axis_name="s"),
        out_type=jax.ShapeDtypeStruct((BATCH, EMBED_DIM), jnp.float32),
        scratch_types=[
            pltpu.VMEM((_B_PER_W,), jnp.int32),
            pltpu.VMEM((_CHUNK, EMBED_DIM), jnp.float32),
            pltpu.SemaphoreType.DMA,
        ],
    )
    def k(table_hbm, idx_hbm, out_hbm, idx_v, rows_v, sem):
        wid = lax.axis_index("s") * _NC + lax.axis_index("c")
        base = wid * _B_PER_W
        pltpu.sync_copy(idx_hbm.at[pl.ds(base, _B_PER_W)], idx_v)
        for c in range(_B_PER_W // _CHUNK):
            off = c * _CHUNK
            pltpu.async_copy(
                table_hbm.at[idx_v.at[pl.ds(off, _CHUNK)]], rows_v, sem
            ).wait()
            pltpu.sync_copy(rows_v, out_hbm.at[pl.ds(base + off, _CHUNK)])

    return k(table, idx)


def _mlp_body(x_ref, w1_ref, b1_ref, g1_ref, be1_ref, w2_ref, b2_ref,
              g2_ref, be2_ref, o_ref):
    x = x_ref[...].astype(jnp.bfloat16)
    h = jnp.dot(x, w1_ref[...].astype(jnp.bfloat16),
                preferred_element_type=jnp.float32)
    h = h + b1_ref[...]
    mu = jnp.mean(h, axis=-1, keepdims=True)
    var = jnp.mean((h - mu) * (h - mu), axis=-1, keepdims=True)
    h = (h - mu) * lax.rsqrt(var + 1e-5) * g1_ref[...] + be1_ref[...]
    h = jnp.maximum(h, 0.0)
    y = jnp.dot(h.astype(jnp.bfloat16), w2_ref[...].astype(jnp.bfloat16),
                preferred_element_type=jnp.float32)
    y = y + b2_ref[...]
    mu2 = jnp.mean(y, axis=-1, keepdims=True)
    var2 = jnp.mean((y - mu2) * (y - mu2), axis=-1, keepdims=True)
    o_ref[...] = (y - mu2) * lax.rsqrt(var2 + 1e-5) * g2_ref[...] + be2_ref[...]


def _tc_mlp(x, W1, b1, g1, be1, W2, b2, g2, be2):
    BT = 256
    grid = (BATCH // BT,)
    full = lambda shape: pl.BlockSpec(shape, lambda i: (0, 0))
    return pl.pallas_call(
        _mlp_body,
        grid=grid,
        in_specs=[
            pl.BlockSpec((BT, EMBED_DIM), lambda i: (i, 0)),
            full((EMBED_DIM, HIDDEN_DIM)),
            full((1, HIDDEN_DIM)),
            full((1, HIDDEN_DIM)),
            full((1, HIDDEN_DIM)),
            full((HIDDEN_DIM, EMBED_DIM)),
            full((1, EMBED_DIM)),
            full((1, EMBED_DIM)),
            full((1, EMBED_DIM)),
        ],
        out_specs=pl.BlockSpec((BT, EMBED_DIM), lambda i: (i, 0)),
        out_shape=jax.ShapeDtypeStruct((BATCH, EMBED_DIM), jnp.float32),
        compiler_params=pltpu.CompilerParams(
            dimension_semantics=("parallel",),
        ),
    )(x, W1, b1.reshape(1, -1), g1.reshape(1, -1), be1.reshape(1, -1),
      W2, b2.reshape(1, -1), g2.reshape(1, -1), be2.reshape(1, -1))


def kernel(user_classes, table, W1, b1, g1, be1, W2, b2, g2, be2):
    gathered = _sc_gather(table, user_classes.astype(jnp.int32))
    return _tc_mlp(gathered, W1, b1, g1, be1, W2, b2, g2, be2)


# speedup vs baseline: 2.8134x; 1.0956x over previous
import functools

import jax
import jax.numpy as jnp
from jax import lax
from jax.experimental import pallas as pl
from jax.experimental.pallas import tpu as pltpu
from jax.experimental.pallas import tpu_sc as plsc

NUM_USERS = 100000
EMBED_DIM = 1152
HIDDEN_DIM = 512
BATCH = 4096

_NC = 2
_NS = 16
_NW = _NC * _NS
_B_PER_W = BATCH // _NW
_CHUNK = 64


def _sc_gather(table, idx):
    @functools.partial(
        pl.kernel,
        mesh=plsc.VectorSubcoreMesh(core_axis_name="c", subcore_axis_name="s"),
        out_type=jax.ShapeDtypeStruct((BATCH, EMBED_DIM), jnp.float32),
        scratch_types=[
            pltpu.VMEM((_B_PER_W,), jnp.int32),
            pltpu.VMEM((_CHUNK, EMBED_DIM), jnp.float32),
            pltpu.SemaphoreType.DMA,
        ],
    )
    def k(table_hbm, idx_hbm, out_hbm, idx_v, rows_v, sem):
        wid = lax.axis_index("s") * _NC + lax.axis_index("c")
        base = wid * _B_PER_W
        pltpu.sync_copy(idx_hbm.at[pl.ds(base, _B_PER_W)], idx_v)
        for c in range(_B_PER_W // _CHUNK):
            off = c * _CHUNK
            pltpu.async_copy(
                table_hbm.at[idx_v.at[pl.ds(off, _CHUNK)]], rows_v, sem
            ).wait()
            pltpu.sync_copy(rows_v, out_hbm.at[pl.ds(base + off, _CHUNK)])

    return k(table, idx)


def _mlp_body(x_ref, w1_ref, b1_ref, g1_ref, be1_ref, w2_ref, b2_ref,
              g2_ref, be2_ref, o_ref):
    x = x_ref[...].astype(jnp.bfloat16)
    h = jnp.dot(x, w1_ref[...].astype(jnp.bfloat16),
                preferred_element_type=jnp.float32)
    h = h + b1_ref[...]
    mu = jnp.mean(h, axis=-1, keepdims=True)
    m2 = jnp.mean(h * h, axis=-1, keepdims=True)
    inv = lax.rsqrt(m2 - mu * mu + 1e-5)
    h = (h - mu) * inv * g1_ref[...] + be1_ref[...]
    h = jnp.maximum(h, 0.0)
    y = jnp.dot(h.astype(jnp.bfloat16), w2_ref[...].astype(jnp.bfloat16),
                preferred_element_type=jnp.float32)
    y = y + b2_ref[...]
    mu2 = jnp.mean(y, axis=-1, keepdims=True)
    m22 = jnp.mean(y * y, axis=-1, keepdims=True)
    inv2 = lax.rsqrt(m22 - mu2 * mu2 + 1e-5)
    o_ref[...] = (y - mu2) * inv2 * g2_ref[...] + be2_ref[...]


def _tc_mlp(x, W1, b1, g1, be1, W2, b2, g2, be2):
    BT = 512
    grid = (BATCH // BT,)
    full = lambda shape: pl.BlockSpec(shape, lambda i: (0, 0))
    return pl.pallas_call(
        _mlp_body,
        grid=grid,
        in_specs=[
            pl.BlockSpec((BT, EMBED_DIM), lambda i: (i, 0)),
            full((EMBED_DIM, HIDDEN_DIM)),
            full((1, HIDDEN_DIM)),
            full((1, HIDDEN_DIM)),
            full((1, HIDDEN_DIM)),
            full((HIDDEN_DIM, EMBED_DIM)),
            full((1, EMBED_DIM)),
            full((1, EMBED_DIM)),
            full((1, EMBED_DIM)),
        ],
        out_specs=pl.BlockSpec((BT, EMBED_DIM), lambda i: (i, 0)),
        out_shape=jax.ShapeDtypeStruct((BATCH, EMBED_DIM), jnp.float32),
        compiler_params=pltpu.CompilerParams(
            dimension_semantics=("parallel",),
        ),
    )(x, W1, b1.reshape(1, -1), g1.reshape(1, -1), be1.reshape(1, -1),
      W2, b2.reshape(1, -1), g2.reshape(1, -1), be2.reshape(1, -1))


def kernel(user_classes, table, W1, b1, g1, be1, W2, b2, g2, be2):
    gathered = _sc_gather(table, user_classes.astype(jnp.int32))
    return _tc_mlp(gathered, W1, b1, g1, be1, W2, b2, g2, be2)


# speedup vs baseline: 2.8494x; 1.0128x over previous
import functools

import jax
import jax.numpy as jnp
from jax import lax
from jax.experimental import pallas as pl
from jax.experimental.pallas import tpu as pltpu
from jax.experimental.pallas import tpu_sc as plsc

NUM_USERS = 100000
EMBED_DIM = 1152
HIDDEN_DIM = 512
BATCH = 4096

_NC = 2
_NS = 16
_NW = _NC * _NS
_CHUNK = 64


def _sc_gather(table, idx, batch):
    b_per_w = batch // _NW

    @functools.partial(
        pl.kernel,
        mesh=plsc.VectorSubcoreMesh(core_axis_name="c", subcore_axis_name="s"),
        out_type=jax.ShapeDtypeStruct((batch, EMBED_DIM), jnp.float32),
        scratch_types=[
            pltpu.VMEM((b_per_w,), jnp.int32),
            pltpu.VMEM((_CHUNK, EMBED_DIM), jnp.float32),
            pltpu.SemaphoreType.DMA,
        ],
    )
    def k(table_hbm, idx_hbm, out_hbm, idx_v, rows_v, sem):
        wid = lax.axis_index("s") * _NC + lax.axis_index("c")
        base = wid * b_per_w
        pltpu.sync_copy(idx_hbm.at[pl.ds(base, b_per_w)], idx_v)
        for c in range(b_per_w // _CHUNK):
            off = c * _CHUNK
            pltpu.async_copy(
                table_hbm.at[idx_v.at[pl.ds(off, _CHUNK)]], rows_v, sem
            ).wait()
            pltpu.sync_copy(rows_v, out_hbm.at[pl.ds(base + off, _CHUNK)])

    return k(table, idx)


def _mlp_math(x_ref, w1_ref, b1_ref, g1_ref, be1_ref, w2_ref, b2_ref,
              g2_ref, be2_ref, o_ref):
    x = x_ref[...].astype(jnp.bfloat16)
    h = jnp.dot(x, w1_ref[...].astype(jnp.bfloat16),
                preferred_element_type=jnp.float32)
    h = h + b1_ref[...]
    mu = jnp.mean(h, axis=-1, keepdims=True)
    m2 = jnp.mean(h * h, axis=-1, keepdims=True)
    inv = lax.rsqrt(m2 - mu * mu + 1e-5)
    h = (h - mu) * inv * g1_ref[...] + be1_ref[...]
    h = jnp.maximum(h, 0.0)
    y = jnp.dot(h.astype(jnp.bfloat16), w2_ref[...].astype(jnp.bfloat16),
                preferred_element_type=jnp.float32)
    y = y + b2_ref[...]
    mu2 = jnp.mean(y, axis=-1, keepdims=True)
    m22 = jnp.mean(y * y, axis=-1, keepdims=True)
    inv2 = lax.rsqrt(m22 - mu2 * mu2 + 1e-5)
    o_ref[...] = (y - mu2) * inv2 * g2_ref[...] + be2_ref[...]


_BT = 512


def _weight_specs():
    full = lambda shape: pl.BlockSpec(shape, lambda i: (0, 0))
    return [
        full((EMBED_DIM, HIDDEN_DIM)),
        full((1, HIDDEN_DIM)),
        full((1, HIDDEN_DIM)),
        full((1, HIDDEN_DIM)),
        full((HIDDEN_DIM, EMBED_DIM)),
        full((1, EMBED_DIM)),
        full((1, EMBED_DIM)),
        full((1, EMBED_DIM)),
    ]


def _tc_mlp_first(x, *weights):
    grid = ((BATCH // 2) // _BT,)
    return pl.pallas_call(
        _mlp_math,
        grid=grid,
        in_specs=[pl.BlockSpec((_BT, EMBED_DIM), lambda i: (i, 0))]
        + _weight_specs(),
        out_specs=pl.BlockSpec((_BT, EMBED_DIM), lambda i: (i, 0)),
        out_shape=jax.ShapeDtypeStruct((BATCH, EMBED_DIM), jnp.float32),
        compiler_params=pltpu.CompilerParams(
            dimension_semantics=("arbitrary",),
        ),
    )(x, *weights)


def _mlp_math_tail(prev_ref, *rest):
    _mlp_math(*rest)


def _tc_mlp_second(prev, x, *weights):
    half_steps = (BATCH // 2) // _BT
    grid = (half_steps,)
    return pl.pallas_call(
        _mlp_math_tail,
        grid=grid,
        in_specs=[
            pl.BlockSpec(memory_space=pl.ANY),
            pl.BlockSpec((_BT, EMBED_DIM), lambda i: (i, 0)),
        ]
        + _weight_specs(),
        out_specs=pl.BlockSpec(
            (_BT, EMBED_DIM), lambda i: (i + half_steps, 0)
        ),
        out_shape=jax.ShapeDtypeStruct((BATCH, EMBED_DIM), jnp.float32),
        input_output_aliases={0: 0},
        compiler_params=pltpu.CompilerParams(
            dimension_semantics=("arbitrary",),
        ),
    )(prev, x, *weights)


def kernel(user_classes, table, W1, b1, g1, be1, W2, b2, g2, be2):
    idx = user_classes.astype(jnp.int32)
    half = BATCH // 2
    g_lo = _sc_gather(table, idx[:half], half)
    g_hi = _sc_gather(table, idx[half:], half)
    weights = (W1, b1.reshape(1, -1), g1.reshape(1, -1), be1.reshape(1, -1),
               W2, b2.reshape(1, -1), g2.reshape(1, -1), be2.reshape(1, -1))
    o_lo = _tc_mlp_first(g_lo, *weights)
    return _tc_mlp_second(o_lo, g_hi, *weights)


# speedup vs baseline: 2.8705x; 1.0074x over previous
import functools

import jax
import jax.numpy as jnp
from jax import lax
from jax.experimental import pallas as pl
from jax.experimental.pallas import tpu as pltpu
from jax.experimental.pallas import tpu_sc as plsc

NUM_USERS = 100000
EMBED_DIM = 1152
HIDDEN_DIM = 512
BATCH = 4096

_NC = 2
_NS = 16
_NW = _NC * _NS
_CHUNK = 64


def _sc_gather(table, idx, offset, batch):
    b_per_w = batch // _NW

    @functools.partial(
        pl.kernel,
        mesh=plsc.VectorSubcoreMesh(core_axis_name="c", subcore_axis_name="s"),
        out_type=jax.ShapeDtypeStruct((batch, EMBED_DIM), jnp.float32),
        scratch_types=[
            pltpu.VMEM((b_per_w,), jnp.int32),
            pltpu.VMEM((_CHUNK, EMBED_DIM), jnp.float32),
            pltpu.SemaphoreType.DMA,
        ],
    )
    def k(table_hbm, idx_hbm, out_hbm, idx_v, rows_v, sem):
        wid = lax.axis_index("s") * _NC + lax.axis_index("c")
        base = wid * b_per_w
        pltpu.sync_copy(idx_hbm.at[pl.ds(offset + base, b_per_w)], idx_v)
        for c in range(b_per_w // _CHUNK):
            off = c * _CHUNK
            pltpu.async_copy(
                table_hbm.at[idx_v.at[pl.ds(off, _CHUNK)]], rows_v, sem
            ).wait()
            pltpu.sync_copy(rows_v, out_hbm.at[pl.ds(base + off, _CHUNK)])

    return k(table, idx)


_BT = 512
_SUB = 256


def _mlp_math(x_ref, w1_ref, b1_ref, g1_ref, be1_ref, w2_ref, b2_ref,
              g2_ref, be2_ref, o_ref):
    for s in range(_BT // _SUB):
        sl = pl.ds(s * _SUB, _SUB)
        x = x_ref[sl, :].astype(jnp.bfloat16)
        h = jnp.dot(x, w1_ref[...], preferred_element_type=jnp.float32)
        h = h + b1_ref[...]
        mu = jnp.mean(h, axis=-1, keepdims=True)
        m2 = jnp.mean(h * h, axis=-1, keepdims=True)
        inv = lax.rsqrt(m2 - mu * mu + 1e-5)
        h = (h - mu) * inv * g1_ref[...] + be1_ref[...]
        h = jnp.maximum(h, 0.0)
        y = jnp.dot(h.astype(jnp.bfloat16), w2_ref[...],
                    preferred_element_type=jnp.float32)
        y = y + b2_ref[...]
        mu2 = jnp.mean(y, axis=-1, keepdims=True)
        m22 = jnp.mean(y * y, axis=-1, keepdims=True)
        inv2 = lax.rsqrt(m22 - mu2 * mu2 + 1e-5)
        o_ref[sl, :] = (y - mu2) * inv2 * g2_ref[...] + be2_ref[...]


def _weight_specs():
    full = lambda shape: pl.BlockSpec(shape, lambda i: (0, 0))
    return [
        full((EMBED_DIM, HIDDEN_DIM)),
        full((1, HIDDEN_DIM)),
        full((1, HIDDEN_DIM)),
        full((1, HIDDEN_DIM)),
        full((HIDDEN_DIM, EMBED_DIM)),
        full((1, EMBED_DIM)),
        full((1, EMBED_DIM)),
        full((1, EMBED_DIM)),
    ]


def _tc_mlp_first(x, *weights):
    grid = ((BATCH // 2) // _BT,)
    return pl.pallas_call(
        _mlp_math,
        grid=grid,
        in_specs=[pl.BlockSpec((_BT, EMBED_DIM), lambda i: (i, 0))]
        + _weight_specs(),
        out_specs=pl.BlockSpec((_BT, EMBED_DIM), lambda i: (i, 0)),
        out_shape=jax.ShapeDtypeStruct((BATCH, EMBED_DIM), jnp.float32),
        compiler_params=pltpu.CompilerParams(
            dimension_semantics=("arbitrary",),
        ),
    )(x, *weights)


def _mlp_math_tail(prev_ref, *rest):
    _mlp_math(*rest)


def _tc_mlp_second(prev, x, *weights):
    half_steps = (BATCH // 2) // _BT
    grid = (half_steps,)
    return pl.pallas_call(
        _mlp_math_tail,
        grid=grid,
        in_specs=[
            pl.BlockSpec(memory_space=pl.ANY),
            pl.BlockSpec((_BT, EMBED_DIM), lambda i: (i, 0)),
        ]
        + _weight_specs(),
        out_specs=pl.BlockSpec(
            (_BT, EMBED_DIM), lambda i: (i + half_steps, 0)
        ),
        out_shape=jax.ShapeDtypeStruct((BATCH, EMBED_DIM), jnp.float32),
        input_output_aliases={0: 0},
        compiler_params=pltpu.CompilerParams(
            dimension_semantics=("arbitrary",),
        ),
    )(prev, x, *weights)


def kernel(user_classes, table, W1, b1, g1, be1, W2, b2, g2, be2):
    idx = user_classes.astype(jnp.int32)
    half = BATCH // 2
    g_lo = _sc_gather(table, idx, 0, half)
    g_hi = _sc_gather(table, idx, half, half)
    weights = (W1.astype(jnp.bfloat16), b1.reshape(1, -1), g1.reshape(1, -1),
               be1.reshape(1, -1), W2.astype(jnp.bfloat16), b2.reshape(1, -1),
               g2.reshape(1, -1), be2.reshape(1, -1))
    o_lo = _tc_mlp_first(g_lo, *weights)
    return _tc_mlp_second(o_lo, g_hi, *weights)
